# baseline (speedup 1.0000x reference)
import functools

import jax
import jax.numpy as jnp
from jax import lax
from jax.experimental import pallas as pl
from jax.experimental.pallas import tpu as pltpu
from jax.experimental.pallas import tpu_sc as plsc

VOCAB = 1000000
EMBED = 32
BATCH = 16384
SEQ = 200

ROWS4 = VOCAB // 4
TC_BLK = 25000
LANES = 16
GROUP = 16
NGROUPS = BATCH // GROUP
IDX_PER_GROUP = GROUP * SEQ
IDX_ROWS = IDX_PER_GROUP // 128
IDX_ROWS_PAD = 32


def _tc_matvec_body(t_ref, wg_ref, o_ref):
    o_ref[...] = jnp.dot(t_ref[...], wg_ref[...],
                         preferred_element_type=jnp.float32)


def _tc_matvec(table4, wg):
    return pl.pallas_call(
        _tc_matvec_body,
        grid=(ROWS4 // TC_BLK,),
        in_specs=[
            pl.BlockSpec((TC_BLK, 128), lambda i: (i, 0)),
            pl.BlockSpec((128, 4), lambda i: (0, 0)),
        ],
        out_specs=pl.BlockSpec((TC_BLK, 4), lambda i: (i, 0)),
        out_shape=jax.ShapeDtypeStruct((ROWS4, 4), jnp.float32),
    )(table4, wg)


def _sc_pool(xt, tw, b16):
    info = plsc.get_sparse_core_info()
    nc, ns = info.num_cores, info.num_subcores
    nw = nc * ns
    per_w = NGROUPS // nw

    @functools.partial(
        pl.kernel,
        out_type=jax.ShapeDtypeStruct((BATCH,), jnp.float32),
        mesh=plsc.VectorSubcoreMesh(core_axis_name="c", subcore_axis_name="s"),
        scratch_types=[
            pltpu.VMEM((IDX_ROWS_PAD, 128), jnp.int32),
            pltpu.VMEM((IDX_ROWS, 128), jnp.float32),
            pltpu.VMEM((LANES,), jnp.float32),
            pltpu.VMEM((LANES,), jnp.float32),
            pltpu.SemaphoreType.DMA,
        ],
    )
    def k(xt_hbm, tw_hbm, b_hbm, out_hbm, idx_v, vals_v, b_v, out_v, sem):
        wid = lax.axis_index("s") * nc + lax.axis_index("c")
        pltpu.sync_copy(b_hbm, b_v)

        def per_group(g, carry):
            gg = wid * per_w + g
            pltpu.sync_copy(
                xt_hbm.at[pl.ds(gg * IDX_ROWS_PAD, IDX_ROWS_PAD)], idx_v)
            cps = []
            for c in range(IDX_ROWS):
                cps.append(pltpu.async_copy(
                    tw_hbm.at[idx_v.at[c]], vals_v.at[c], sem))
            for cp in cps:
                cp.wait()
            acc = jnp.zeros((LANES,), jnp.float32)
            for s in range(SEQ):
                acc = acc + vals_v[s >> 3, pl.ds((s & 7) * LANES, LANES)]
            z = acc * (1.0 / SEQ) + b_v[...]
            out_v[...] = 1.0 / (1.0 + jnp.exp(-z))
            pltpu.sync_copy(out_v, out_hbm.at[pl.ds(gg * GROUP, GROUP)])
            return carry

        lax.fori_loop(0, per_w, per_group, 0)

    return k(xt, tw, b16)


def kernel(x, table, W, b):
    table4 = table.reshape(ROWS4, 128)
    wg = jnp.kron(jnp.eye(4, dtype=jnp.float32), W)
    tw = _tc_matvec(table4, wg).reshape(VOCAB)
    xt = (x.astype(jnp.int32)
           .reshape(NGROUPS, GROUP, SEQ)
           .transpose(0, 2, 1)
           .reshape(NGROUPS, IDX_ROWS, 128))
    xt = jnp.pad(xt, ((0, 0), (0, IDX_ROWS_PAD - IDX_ROWS), (0, 0)))
    xt = xt.reshape(NGROUPS * IDX_ROWS_PAD, 128)
    b16 = jnp.broadcast_to(b.astype(jnp.float32), (LANES,))
    return _sc_pool(xt, tw, b16)


# speedup vs baseline: 1.1844x; 1.1844x over previous
import functools

import jax
import jax.numpy as jnp
from jax import lax
from jax.experimental import pallas as pl
from jax.experimental.pallas import tpu as pltpu
from jax.experimental.pallas import tpu_sc as plsc

VOCAB = 1000000
EMBED = 32
BATCH = 16384
SEQ = 200

ROWS4 = VOCAB // 4
TC_BLK = 25000
LANES = 16
GROUP = 16
NGROUPS = BATCH // GROUP
TAIL = SEQ - 128


def _tc_matvec_body(t_ref, wg_ref, b_ref, o_ref):
    o_ref[...] = jnp.dot(t_ref[...], wg_ref[...],
                         preferred_element_type=jnp.float32) + b_ref[0]


def _tc_matvec(table4, wg, b):
    return pl.pallas_call(
        _tc_matvec_body,
        grid=(ROWS4 // TC_BLK,),
        in_specs=[
            pl.BlockSpec((TC_BLK, 128), lambda i: (i, 0)),
            pl.BlockSpec((128, 4), lambda i: (0, 0)),
            pl.BlockSpec(memory_space=pltpu.SMEM),
        ],
        out_specs=pl.BlockSpec((TC_BLK, 4), lambda i: (i, 0)),
        out_shape=jax.ShapeDtypeStruct((ROWS4, 4), jnp.float32),
    )(table4, wg, b)


def _sc_pool(x, tw):
    info = plsc.get_sparse_core_info()
    nc, ns = info.num_cores, info.num_subcores
    nw = nc * ns
    per_w = NGROUPS // nw

    @functools.partial(
        pl.kernel,
        out_type=jax.ShapeDtypeStruct((BATCH,), jnp.float32),
        mesh=plsc.VectorSubcoreMesh(core_axis_name="c", subcore_axis_name="s"),
        scratch_types=[
            pltpu.VMEM_SHARED((VOCAB,), jnp.float32),
            pltpu.VMEM_SHARED((16 * 256,), jnp.float32),
            pltpu.VMEM((GROUP, SEQ), jnp.int32),
            pltpu.VMEM((GROUP, 208), jnp.float32),
            pltpu.VMEM((256,), jnp.float32),
            pltpu.VMEM((256,), jnp.int32),
            pltpu.VMEM((256,), jnp.float32),
            pltpu.VMEM((LANES,), jnp.float32),
            pltpu.SemaphoreType.DMA,
        ],
    )
    def k(x_hbm, tw_hbm, out_hbm, tw_sh, xp_sh, i_v, a_v, pacc_v, patt_v,
          accT_v, out_v, sem):
        cid = lax.axis_index("c")
        sid = lax.axis_index("s")
        wid = sid * nc + cid

        @pl.when(sid == 0)
        def _stage():
            pltpu.sync_copy(tw_hbm, tw_sh)
        plsc.subcore_barrier()

        base = sid * 256
        for a in range(LANES):
            patt_v[pl.ds(a * LANES, LANES)] = (
                lax.iota(jnp.int32, LANES) * LANES + (base + a))

        def per_group(g, carry):
            gg = wid * per_w + g
            pltpu.sync_copy(x_hbm.at[pl.ds(gg * GROUP, GROUP), :], i_v)
            zeros16 = jnp.zeros((LANES,), jnp.float32)
            for r in range(GROUP):
                a_v[r, pl.ds(192, 16)] = zeros16
            cps = []
            for r in range(GROUP):
                cps.append(pltpu.async_copy(
                    tw_sh.at[i_v.at[r, pl.ds(0, 128)]],
                    a_v.at[r, pl.ds(0, 128)], sem))
                cps.append(pltpu.async_copy(
                    tw_sh.at[i_v.at[r, pl.ds(128, TAIL)]],
                    a_v.at[r, pl.ds(128, TAIL)], sem))
            for cp in cps:
                cp.wait()
            for r in range(GROUP):
                p = a_v[r, pl.ds(0, 16)]
                for c in range(1, 13):
                    p = p + a_v[r, pl.ds(c * 16, 16)]
                pacc_v[pl.ds(r * LANES, LANES)] = p
            pltpu.sync_copy(pacc_v, xp_sh.at[pl.ds(base, 256)])
            g0 = pltpu.async_copy(xp_sh.at[patt_v.at[pl.ds(0, 128)]],
                                  accT_v.at[pl.ds(0, 128)], sem)
            g1 = pltpu.async_copy(xp_sh.at[patt_v.at[pl.ds(128, 128)]],
                                  accT_v.at[pl.ds(128, 128)], sem)
            g0.wait()
            g1.wait()
            tot = accT_v[pl.ds(0, 16)]
            for a in range(1, LANES):
                tot = tot + accT_v[pl.ds(a * LANES, LANES)]
            z = tot * (1.0 / SEQ)
            out_v[...] = 1.0 / (1.0 + jnp.exp(-z))
            pltpu.sync_copy(out_v, out_hbm.at[pl.ds(gg * GROUP, GROUP)])
            return carry

        lax.fori_loop(0, per_w, per_group, 0)

    return k(x, tw)


def kernel(x, table, W, b):
    table4 = table.reshape(ROWS4, 128)
    wg = jnp.kron(jnp.eye(4, dtype=jnp.float32), W)
    tw = _tc_matvec(table4, wg, b.astype(jnp.float32)).reshape(VOCAB)
    return _sc_pool(x.astype(jnp.int32), tw)


# speedup vs baseline: 4.6893x; 3.9590x over previous
import functools

import jax
import jax.numpy as jnp
from jax import lax
from jax.experimental import pallas as pl
from jax.experimental.pallas import tpu as pltpu
from jax.experimental.pallas import tpu_sc as plsc

VOCAB = 1000000
EMBED = 32
BATCH = 16384
SEQ = 200

TC_BLK = 131072
LANES = 16
GROUP = 16
NGROUPS = BATCH // GROUP
TAIL = SEQ - 128


def _tc_matvec_body(a_ref, t_ref, o_ref):
    o_ref[...] = jnp.dot(a_ref[...], t_ref[...],
                         preferred_element_type=jnp.float32)


def _tc_matvec(a8, table_t):
    return pl.pallas_call(
        _tc_matvec_body,
        grid=(pl.cdiv(VOCAB, TC_BLK),),
        in_specs=[
            pl.BlockSpec((8, EMBED), lambda i: (0, 0)),
            pl.BlockSpec((EMBED, TC_BLK), lambda i: (0, i)),
        ],
        out_specs=pl.BlockSpec((8, TC_BLK), lambda i: (0, i)),
        out_shape=jax.ShapeDtypeStruct((8, VOCAB), jnp.float32),
    )(a8, table_t)


def _sc_pool(x, tw):
    info = plsc.get_sparse_core_info()
    nc, ns = info.num_cores, info.num_subcores
    nw = nc * ns
    per_w = NGROUPS // nw

    @functools.partial(
        pl.kernel,
        out_type=jax.ShapeDtypeStruct((BATCH,), jnp.float32),
        mesh=plsc.VectorSubcoreMesh(core_axis_name="c", subcore_axis_name="s"),
        scratch_types=[
            pltpu.VMEM_SHARED((VOCAB,), jnp.float32),
            pltpu.VMEM_SHARED((16 * 256,), jnp.float32),
            pltpu.VMEM((GROUP, SEQ), jnp.int32),
            pltpu.VMEM((GROUP, 208), jnp.float32),
            pltpu.VMEM((256,), jnp.float32),
            pltpu.VMEM((256,), jnp.int32),
            pltpu.VMEM((256,), jnp.float32),
            pltpu.VMEM((LANES,), jnp.float32),
            pltpu.SemaphoreType.DMA,
        ],
    )
    def k(x_hbm, tw_hbm, out_hbm, tw_sh, xp_sh, i_v, a_v, pacc_v, patt_v,
          accT_v, out_v, sem):
        cid = lax.axis_index("c")
        sid = lax.axis_index("s")
        wid = sid * nc + cid

        @pl.when(sid == 0)
        def _stage():
            pltpu.sync_copy(tw_hbm, tw_sh)
        plsc.subcore_barrier()

        base = sid * 256
        for a in range(LANES):
            patt_v[pl.ds(a * LANES, LANES)] = (
                lax.iota(jnp.int32, LANES) * LANES + (base + a))

        def per_group(g, carry):
            gg = wid * per_w + g
            pltpu.sync_copy(x_hbm.at[pl.ds(gg * GROUP, GROUP), :], i_v)
            zeros16 = jnp.zeros((LANES,), jnp.float32)
            for r in range(GROUP):
                a_v[r, pl.ds(192, 16)] = zeros16
            cps = []
            for r in range(GROUP):
                cps.append(pltpu.async_copy(
                    tw_sh.at[i_v.at[r, pl.ds(0, 128)]],
                    a_v.at[r, pl.ds(0, 128)], sem))
                cps.append(pltpu.async_copy(
                    tw_sh.at[i_v.at[r, pl.ds(128, TAIL)]],
                    a_v.at[r, pl.ds(128, TAIL)], sem))
            for cp in cps:
                cp.wait()
            for r in range(GROUP):
                p = a_v[r, pl.ds(0, 16)]
                for c in range(1, 13):
                    p = p + a_v[r, pl.ds(c * 16, 16)]
                pacc_v[pl.ds(r * LANES, LANES)] = p
            pltpu.sync_copy(pacc_v, xp_sh.at[pl.ds(base, 256)])
            g0 = pltpu.async_copy(xp_sh.at[patt_v.at[pl.ds(0, 128)]],
                                  accT_v.at[pl.ds(0, 128)], sem)
            g1 = pltpu.async_copy(xp_sh.at[patt_v.at[pl.ds(128, 128)]],
                                  accT_v.at[pl.ds(128, 128)], sem)
            g0.wait()
            g1.wait()
            tot = accT_v[pl.ds(0, 16)]
            for a in range(1, LANES):
                tot = tot + accT_v[pl.ds(a * LANES, LANES)]
            z = tot * (1.0 / SEQ)
            out_v[...] = 1.0 / (1.0 + jnp.exp(-z))
            pltpu.sync_copy(out_v, out_hbm.at[pl.ds(gg * GROUP, GROUP)])
            return carry

        lax.fori_loop(0, per_w, per_group, 0)

    return k(x, tw)


def kernel(x, table, W, b):
    table_t = jnp.transpose(table)
    a8 = jnp.zeros((8, EMBED), jnp.float32).at[0].set(W[:, 0])
    out8 = _tc_matvec(a8, table_t)
    tw = out8.sum(axis=0) + b[0].astype(jnp.float32)
    return _sc_pool(x.astype(jnp.int32), tw)


# speedup vs baseline: 5.8894x; 1.2559x over previous
import functools

import jax
import jax.numpy as jnp
from jax import lax
from jax.experimental import pallas as pl
from jax.experimental.pallas import tpu as pltpu
from jax.experimental.pallas import tpu_sc as plsc

VOCAB = 1000000
EMBED = 32
BATCH = 16384
SEQ = 200

TC_BLK = 131072
LANES = 16
GROUP = 16
NGROUPS = BATCH // GROUP
TAIL = SEQ - 128


VPAD = TC_BLK * 8


def _tc_matvec_body(a_ref, b_ref, t_ref, o_ref):
    i = pl.program_id(0)
    r8 = jnp.dot(a_ref[...], t_ref[...],
                 preferred_element_type=jnp.float32)
    o_ref[pl.ds(i * TC_BLK, TC_BLK)] = r8.sum(axis=0) + b_ref[0]


def _tc_matvec(a8, table_t, b):
    return pl.pallas_call(
        _tc_matvec_body,
        grid=(VPAD // TC_BLK,),
        in_specs=[
            pl.BlockSpec((8, EMBED), lambda i: (0, 0)),
            pl.BlockSpec(memory_space=pltpu.SMEM),
            pl.BlockSpec((EMBED, TC_BLK), lambda i: (0, i)),
        ],
        out_specs=pl.BlockSpec((VPAD,), lambda i: (0,)),
        out_shape=jax.ShapeDtypeStruct((VPAD,), jnp.float32),
    )(a8, b, table_t)


def _sc_pool(x, tw):
    info = plsc.get_sparse_core_info()
    nc, ns = info.num_cores, info.num_subcores
    nw = nc * ns
    per_w = NGROUPS // nw

    @functools.partial(
        pl.kernel,
        out_type=jax.ShapeDtypeStruct((BATCH,), jnp.float32),
        mesh=plsc.VectorSubcoreMesh(core_axis_name="c", subcore_axis_name="s"),
        scratch_types=[
            pltpu.VMEM_SHARED((VPAD,), jnp.float32),
            pltpu.VMEM_SHARED((16 * 256,), jnp.float32),
            pltpu.VMEM((GROUP, SEQ), jnp.int32),
            pltpu.VMEM((GROUP, 208), jnp.float32),
            pltpu.VMEM((256,), jnp.float32),
            pltpu.VMEM((256,), jnp.int32),
            pltpu.VMEM((256,), jnp.float32),
            pltpu.VMEM((LANES,), jnp.float32),
            pltpu.SemaphoreType.DMA,
        ],
    )
    def k(x_hbm, tw_hbm, out_hbm, tw_sh, xp_sh, i_v, a_v, pacc_v, patt_v,
          accT_v, out_v, sem):
        cid = lax.axis_index("c")
        sid = lax.axis_index("s")
        wid = sid * nc + cid

        @pl.when(sid == 0)
        def _stage():
            pltpu.sync_copy(tw_hbm, tw_sh)
        plsc.subcore_barrier()

        base = sid * 256
        for a in range(LANES):
            patt_v[pl.ds(a * LANES, LANES)] = (
                lax.iota(jnp.int32, LANES) * LANES + (base + a))

        def per_group(g, carry):
            gg = wid * per_w + g
            pltpu.sync_copy(x_hbm.at[pl.ds(gg * GROUP, GROUP), :], i_v)
            zeros16 = jnp.zeros((LANES,), jnp.float32)
            for r in range(GROUP):
                a_v[r, pl.ds(192, 16)] = zeros16
            cps = []
            for r in range(GROUP):
                cps.append(pltpu.async_copy(
                    tw_sh.at[i_v.at[r, pl.ds(0, 128)]],
                    a_v.at[r, pl.ds(0, 128)], sem))
                cps.append(pltpu.async_copy(
                    tw_sh.at[i_v.at[r, pl.ds(128, TAIL)]],
                    a_v.at[r, pl.ds(128, TAIL)], sem))
            for cp in cps:
                cp.wait()
            for r in range(GROUP):
                p = a_v[r, pl.ds(0, 16)]
                for c in range(1, 13):
                    p = p + a_v[r, pl.ds(c * 16, 16)]
                pacc_v[pl.ds(r * LANES, LANES)] = p
            pltpu.sync_copy(pacc_v, xp_sh.at[pl.ds(base, 256)])
            g0 = pltpu.async_copy(xp_sh.at[patt_v.at[pl.ds(0, 128)]],
                                  accT_v.at[pl.ds(0, 128)], sem)
            g1 = pltpu.async_copy(xp_sh.at[patt_v.at[pl.ds(128, 128)]],
                                  accT_v.at[pl.ds(128, 128)], sem)
            g0.wait()
            g1.wait()
            tot = accT_v[pl.ds(0, 16)]
            for a in range(1, LANES):
                tot = tot + accT_v[pl.ds(a * LANES, LANES)]
            z = tot * (1.0 / SEQ)
            out_v[...] = 1.0 / (1.0 + jnp.exp(-z))
            pltpu.sync_copy(out_v, out_hbm.at[pl.ds(gg * GROUP, GROUP)])
            return carry

        lax.fori_loop(0, per_w, per_group, 0)

    return k(x, tw)


def kernel(x, table, W, b):
    table_t = jnp.transpose(table)
    a8 = jnp.zeros((8, EMBED), jnp.float32).at[0].set(W[:, 0])
    tw = _tc_matvec(a8, table_t, b.astype(jnp.float32))
    return _sc_pool(x.astype(jnp.int32), tw)


# speedup vs baseline: 6.2190x; 1.0560x over previous
import functools

import jax
import jax.numpy as jnp
from jax import lax
from jax.experimental import pallas as pl
from jax.experimental.pallas import tpu as pltpu
from jax.experimental.pallas import tpu_sc as plsc

VOCAB = 1000000
EMBED = 32
BATCH = 16384
SEQ = 200

TC_BLK = 131072
LANES = 16
GROUP = 16
NGROUPS = BATCH // GROUP
TAIL = SEQ - 128


VPAD = TC_BLK * 8


def _tc_matvec_body(a_ref, b_ref, t_ref, o_ref):
    i = pl.program_id(0)
    r8 = jnp.dot(a_ref[...], t_ref[...],
                 preferred_element_type=jnp.float32)
    o_ref[pl.ds(i * TC_BLK, TC_BLK)] = r8.sum(axis=0) + b_ref[0]


def _tc_matvec(a8, table_t, b):
    return pl.pallas_call(
        _tc_matvec_body,
        grid=(VPAD // TC_BLK,),
        in_specs=[
            pl.BlockSpec((8, EMBED), lambda i: (0, 0)),
            pl.BlockSpec(memory_space=pltpu.SMEM),
            pl.BlockSpec((EMBED, TC_BLK), lambda i: (0, i)),
        ],
        out_specs=pl.BlockSpec((VPAD,), lambda i: (0,)),
        out_shape=jax.ShapeDtypeStruct((VPAD,), jnp.float32),
    )(a8, b, table_t)


def _rot(x, idx):
    return jax.lax.gather(
        x, idx[:, None],
        jax.lax.GatherDimensionNumbers(
            offset_dims=(), collapsed_slice_dims=(0,), start_index_map=(0,)),
        (1,), mode=jax.lax.GatherScatterMode.PROMISE_IN_BOUNDS)


def _sc_pool(x, tw):
    info = plsc.get_sparse_core_info()
    nc, ns = info.num_cores, info.num_subcores
    nw = nc * ns
    per_w = NGROUPS // nw

    @functools.partial(
        pl.kernel,
        out_type=jax.ShapeDtypeStruct((BATCH,), jnp.float32),
        mesh=plsc.VectorSubcoreMesh(core_axis_name="c", subcore_axis_name="s"),
        scratch_types=[
            pltpu.VMEM_SHARED((VPAD,), jnp.float32),
            pltpu.VMEM((GROUP, SEQ), jnp.int32),
            pltpu.VMEM((GROUP, 208), jnp.float32),
            pltpu.VMEM((LANES,), jnp.float32),
            pltpu.SemaphoreType.DMA,
        ],
    )
    def k(x_hbm, tw_hbm, out_hbm, tw_sh, i_v, a_v, out_v, sem):
        cid = lax.axis_index("c")
        sid = lax.axis_index("s")
        wid = sid * nc + cid

        @pl.when(sid == 0)
        def _stage():
            pltpu.sync_copy(tw_hbm, tw_sh)
        plsc.subcore_barrier()

        def per_group(g, carry):
            gg = wid * per_w + g
            pltpu.sync_copy(x_hbm.at[pl.ds(gg * GROUP, GROUP), :], i_v)
            zeros16 = jnp.zeros((LANES,), jnp.float32)
            for r in range(GROUP):
                a_v[r, pl.ds(192, 16)] = zeros16
            cps = []
            for r in range(GROUP):
                cps.append(pltpu.async_copy(
                    tw_sh.at[i_v.at[r, pl.ds(0, 128)]],
                    a_v.at[r, pl.ds(0, 128)], sem))
                cps.append(pltpu.async_copy(
                    tw_sh.at[i_v.at[r, pl.ds(128, TAIL)]],
                    a_v.at[r, pl.ds(128, TAIL)], sem))
            for cp in cps:
                cp.wait()
            vs = []
            for r in range(GROUP):
                p = a_v[r, pl.ds(0, 16)]
                for c in range(1, 13):
                    p = p + a_v[r, pl.ds(c * 16, 16)]
                vs.append(p)
            lanes = lax.iota(jnp.int32, LANES)
            for kk in (1, 2, 4, 8):
                m = (lanes & kk) == 0
                ip = (lanes + kk) & (LANES - 1)
                im = (lanes - kk) & (LANES - 1)
                nvs = []
                for j in range(len(vs) // 2):
                    a, b = vs[2 * j], vs[2 * j + 1]
                    w = (jnp.where(m, a, _rot(b, im)) +
                         jnp.where(m, _rot(a, ip), b))
                    nvs.append(w)
                vs = nvs
            z = vs[0] * (1.0 / SEQ)
            out_v[...] = 1.0 / (1.0 + jnp.exp(-z))
            pltpu.sync_copy(out_v, out_hbm.at[pl.ds(gg * GROUP, GROUP)])
            return carry

        lax.fori_loop(0, per_w, per_group, 0)

    return k(x, tw)


def kernel(x, table, W, b):
    table_t = jnp.transpose(table)
    a8 = jnp.zeros((8, EMBED), jnp.float32).at[0].set(W[:, 0])
    tw = _tc_matvec(a8, table_t, b.astype(jnp.float32))
    return _sc_pool(x.astype(jnp.int32), tw)


# speedup vs baseline: 6.2262x; 1.0012x over previous
import functools

import jax
import jax.numpy as jnp
from jax import lax
from jax.experimental import pallas as pl
from jax.experimental.pallas import tpu as pltpu
from jax.experimental.pallas import tpu_sc as plsc

VOCAB = 1000000
EMBED = 32
BATCH = 16384
SEQ = 200

TC_BLK = 131072
LANES = 16
GROUP = 16
NGROUPS = BATCH // GROUP
TAIL = SEQ - 128


VPAD = TC_BLK * 8


def _tc_matvec_body(a_ref, b_ref, t_ref, o_ref):
    i = pl.program_id(0)
    r8 = jnp.dot(a_ref[...], t_ref[...],
                 preferred_element_type=jnp.float32)
    o_ref[pl.ds(i * TC_BLK, TC_BLK)] = r8.sum(axis=0) + b_ref[0]


def _tc_matvec(a8, table_t, b):
    return pl.pallas_call(
        _tc_matvec_body,
        grid=(VPAD // TC_BLK,),
        in_specs=[
            pl.BlockSpec((8, EMBED), lambda i: (0, 0)),
            pl.BlockSpec(memory_space=pltpu.SMEM),
            pl.BlockSpec((EMBED, TC_BLK), lambda i: (0, i)),
        ],
        out_specs=pl.BlockSpec((VPAD,), lambda i: (0,)),
        out_shape=jax.ShapeDtypeStruct((VPAD,), jnp.float32),
    )(a8, b, table_t)


def _rot(x, idx):
    return jax.lax.gather(
        x, idx[:, None],
        jax.lax.GatherDimensionNumbers(
            offset_dims=(), collapsed_slice_dims=(0,), start_index_map=(0,)),
        (1,), mode=jax.lax.GatherScatterMode.PROMISE_IN_BOUNDS)


def _sc_pool(x, tw):
    info = plsc.get_sparse_core_info()
    nc, ns = info.num_cores, info.num_subcores
    nw = nc * ns
    per_w = NGROUPS // nw

    @functools.partial(
        pl.kernel,
        out_type=jax.ShapeDtypeStruct((BATCH,), jnp.float32),
        mesh=plsc.VectorSubcoreMesh(core_axis_name="c", subcore_axis_name="s"),
        compiler_params=pltpu.CompilerParams(use_tc_tiling_on_sc=True),
        scratch_types=[
            pltpu.VMEM_SHARED((VPAD,), jnp.float32),
            pltpu.VMEM((GROUP, SEQ), jnp.int32),
            pltpu.VMEM((GROUP, 208), jnp.float32),
            pltpu.VMEM((LANES,), jnp.float32),
            pltpu.SemaphoreType.DMA,
        ],
    )
    def k(x_hbm, tw_hbm, out_hbm, tw_sh, i_v, a_v, out_v, sem):
        cid = lax.axis_index("c")
        sid = lax.axis_index("s")
        wid = sid * nc + cid

        @pl.when(sid == 0)
        def _stage():
            pltpu.sync_copy(tw_hbm, tw_sh)
        plsc.subcore_barrier()

        def per_group(g, carry):
            gg = wid * per_w + g
            pltpu.sync_copy(x_hbm.at[pl.ds(gg * GROUP, GROUP), :], i_v)
            zeros16 = jnp.zeros((LANES,), jnp.float32)
            for r in range(GROUP):
                a_v[r, pl.ds(192, 16)] = zeros16
            cps = []
            for r in range(GROUP):
                cps.append(pltpu.async_copy(
                    tw_sh.at[i_v.at[r, pl.ds(0, 128)]],
                    a_v.at[r, pl.ds(0, 128)], sem))
                cps.append(pltpu.async_copy(
                    tw_sh.at[i_v.at[r, pl.ds(128, TAIL)]],
                    a_v.at[r, pl.ds(128, TAIL)], sem))
            for cp in cps:
                cp.wait()
            vs = []
            for r in range(GROUP):
                p = a_v[r, pl.ds(0, 16)]
                for c in range(1, 13):
                    p = p + a_v[r, pl.ds(c * 16, 16)]
                vs.append(p)
            lanes = lax.iota(jnp.int32, LANES)
            for kk in (1, 2, 4, 8):
                m = (lanes & kk) == 0
                ip = (lanes + kk) & (LANES - 1)
                im = (lanes - kk) & (LANES - 1)
                nvs = []
                for j in range(len(vs) // 2):
                    a, b = vs[2 * j], vs[2 * j + 1]
                    w = (jnp.where(m, a, _rot(b, im)) +
                         jnp.where(m, _rot(a, ip), b))
                    nvs.append(w)
                vs = nvs
            z = vs[0] * (1.0 / SEQ)
            out_v[...] = 1.0 / (1.0 + jnp.exp(-z))
            pltpu.sync_copy(out_v, out_hbm.at[pl.ds(gg * GROUP, GROUP)])
            return carry

        lax.fori_loop(0, per_w, per_group, 0)

    return k(x, tw)


def kernel(x, table, W, b):
    table_t = jnp.transpose(table)
    a8 = jnp.zeros((8, EMBED), jnp.float32).at[0].set(W[:, 0])
    tw = _tc_matvec(a8, table_t, b.astype(jnp.float32))
    return _sc_pool(x.astype(jnp.int32), tw)
